# baseline (speedup 1.0000x reference)
import functools

import numpy as np
import jax
import jax.numpy as jnp
from jax import lax
from jax.experimental import pallas as pl
from jax.experimental.pallas import tpu as pltpu
from jax.experimental.pallas import tpu_sc as plsc

N = 4096; E = 65536; T = 3; C = 8; D = 128; NH = 8; DH = 16; PH = 64
TN = T * N
TE = T * E
NC = 2
NS = 16
NW = NC * NS
K = 64
EPW = E // NW
NCH = EPW // K
BR = 256
NB = N // BR
NBT = TN // BR
BR2 = 512
F32 = jnp.float32

_PERM = np.array([(k % NH) * DH + k // NH for k in range(D)], dtype=np.int32)

_mesh = plsc.VectorSubcoreMesh(core_axis_name="c", subcore_axis_name="s",
                               num_cores=NC, num_subcores=NS)


NP = N // 8
KG = 32
NCHG = EPW // KG


def _gat_edge_body(hp_h, ab_h, m3_h, src_h, dst_h, ew_h, aggo_h, deno_h,
                   agg_s, den_s, m3b, denl, idb,
                   sb0, db0, dba0, dx0, ew0, hr0, as0, ad0,
                   sb1, db1, dba1, dx1, ew1, hr1, as1, ad1,
                   sem0, sem1):
    cid = lax.axis_index("c")
    sid = lax.axis_index("s")
    wid = sid * NC + cid
    rows = N // NS
    r0 = sid * rows
    prow = sid * (NP // NS)
    sets = ((sb0, db0, dba0, dx0, ew0, hr0, as0, ad0, sem0),
            (sb1, db1, dba1, dx1, ew1, hr1, as1, ad1, sem1))

    pltpu.sync_copy(m3_h, m3b)
    for j in range(NP // 16):
        idb[pl.ds(j * 16, 16)] = (lax.iota(jnp.int32, 16) + j * 16)

    def loadidx(t, ch, st, off):
        sb, db, dba, dx, ewv = st[0], st[1], st[2], st[3], st[4]
        base = t * E + wid * EPW + ch * KG
        pltpu.sync_copy(src_h.at[pl.ds(base, KG)], sb)
        pltpu.sync_copy(dst_h.at[pl.ds(base, KG)], db)
        pltpu.sync_copy(ew_h.at[pl.ds(base, KG)], ewv.at[pl.ds(0, KG)])
        for j in range(KG // 16):
            sl = pl.ds(j * 16, 16)
            dv = db[sl]
            sb[sl] = sb[sl] + off
            dba[sl] = dv + off
            dx[sl] = dv
        pltpu.async_copy(hp_h.at[sb], st[5], st[8])
        pltpu.async_copy(ab_h.at[sb], st[6], st[8])
        pltpu.async_copy(ab_h.at[dba], st[7], st[8])

    def compute(st, mv):
        db, dx, ewv, hr, as_, ad_ = st[1], st[3], st[4], st[5], st[6], st[7]
        pltpu.make_async_copy(hp_h.at[st[0]], hr, st[8]).wait()
        pltpu.make_async_copy(ab_h.at[st[0]], as_, st[8]).wait()
        pltpu.make_async_copy(ab_h.at[st[2]], ad_, st[8]).wait()

        def edge(i, _):
            va = as_[i, pl.ds(0, 16)]
            vb = ad_[i, pl.ds(16, 16)]
            e = va + vb
            e = jnp.where(e > 0, e, 0.2 * e)
            ex = jnp.exp(e - mv) * ewv[pl.ds(i, 16)][0]
            dl = dx[pl.ds(i, 16)][0]
            row = lax.shift_right_logical(dl, 3)
            loff = (dl & 7) * 16
            denl[row, pl.ds(loff, 16)] = denl[row, pl.ds(loff, 16)] + ex
            for r in range(8):
                sl = pl.ds(r * 16, 16)
                hr[i, sl] = hr[i, sl] * ex
            return 0
        lax.fori_loop(0, KG, edge, 0)
        pltpu.sync_copy(hr, agg_s.at[db], add=True)

    for t in range(T):
        def zb(i, _):
            for r in range(8):
                hr0[i, pl.ds(r * 16, 16)] = jnp.zeros((16,), F32)
            return 0
        lax.fori_loop(0, KG, zb, 0)

        def zd(i, _):
            for r in range(8):
                denl[i, pl.ds(r * 16, 16)] = jnp.zeros((16,), F32)
            return 0
        lax.fori_loop(0, NP, zd, 0)
        for j in range(rows // KG):
            pltpu.sync_copy(hr0, agg_s.at[pl.ds(r0 + j * KG, KG)])
        pltpu.sync_copy(hr0.at[pl.ds(0, NP // NS)],
                        den_s.at[pl.ds(prow, NP // NS)])
        plsc.subcore_barrier()

        mv = m3b[t, pl.ds(0, 16)]
        off = jnp.full((16,), t * N, jnp.int32)
        loadidx(t, 0, sets[0], off)

        def pair(c2, _):
            c0 = 2 * c2
            loadidx(t, c0 + 1, sets[1], off)
            compute(sets[0], mv)

            @pl.when(c0 + 2 < NCHG)
            def _():
                loadidx(t, c0 + 2, sets[0], off)
            compute(sets[1], mv)
            return 0
        lax.fori_loop(0, NCHG // 2, pair, 0)

        pltpu.sync_copy(denl, den_s.at[idb], add=True)
        plsc.subcore_barrier()
        pltpu.sync_copy(agg_s.at[pl.ds(r0, rows)],
                        aggo_h.at[pl.ds(cid * TN + t * N + r0, rows)])
        pltpu.sync_copy(den_s.at[pl.ds(prow, NP // NS)],
                        deno_h.at[pl.ds(cid * T * NP + t * NP + prow,
                                        NP // NS)])
        plsc.subcore_barrier()


def _gat_scratch():
    per_set = [
        pltpu.VMEM((KG,), jnp.int32),
        pltpu.VMEM((KG,), jnp.int32),
        pltpu.VMEM((KG,), jnp.int32),
        pltpu.VMEM((KG + 16,), jnp.int32),
        pltpu.VMEM((KG + 16,), F32),
        pltpu.VMEM((KG, D), F32),
        pltpu.VMEM((KG, D), F32),
        pltpu.VMEM((KG, D), F32),
    ]
    return ([pltpu.VMEM_SHARED((N, D), F32),
             pltpu.VMEM_SHARED((NP, D), F32),
             pltpu.VMEM((8, D), F32),
             pltpu.VMEM((NP, D), F32),
             pltpu.VMEM((NP,), jnp.int32)]
            + per_set + per_set
            + [pltpu.SemaphoreType.DMA, pltpu.SemaphoreType.DMA])


_gat_edge = pl.kernel(
    _gat_edge_body,
    out_type=[jax.ShapeDtypeStruct((2 * TN, D), F32),
              jax.ShapeDtypeStruct((2 * T * NP, D), F32)],
    mesh=_mesh,
    scratch_types=_gat_scratch(),
)


def _prop_body(x_h, pp_h, src_h, dst_h, ew_h, aggg_h, aggm_h,
               acc_s, gbuf,
               sb0, db0, dba0, ew0, xr0, pr0, dr0,
               sb1, db1, dba1, ew1, xr1, pr1, dr1,
               sem0, sem1):
    cid = lax.axis_index("c")
    sid = lax.axis_index("s")
    wid = sid * NC + cid
    rows = N // NS
    r0 = sid * rows
    sets = ((sb0, db0, dba0, ew0, xr0, pr0, dr0, sem0),
            (sb1, db1, dba1, ew1, xr1, pr1, dr1, sem1))

    def loadidx(t, ch, st, off, pm):
        sb, db, dba, ewv = st[0], st[1], st[2], st[3]
        base = t * E + wid * EPW + ch * K
        pltpu.sync_copy(src_h.at[pl.ds(base, K)], sb)
        pltpu.sync_copy(dst_h.at[pl.ds(base, K)], db)
        pltpu.sync_copy(ew_h.at[pl.ds(base, K)], ewv.at[pl.ds(0, K)])
        for j in range(K // 16):
            sl = pl.ds(j * 16, 16)
            sb[sl] = sb[sl] + off
            dba[sl] = db[sl] + off
        pltpu.async_copy(x_h.at[sb], st[4], st[7])
        if pm == 1:
            pltpu.async_copy(pp_h.at[sb], st[5], st[7])
            pltpu.async_copy(pp_h.at[dba], st[6], st[7])

    def compute(st, pm):
        db, ewv, xr, pr, dr = st[1], st[3], st[4], st[5], st[6]
        pltpu.make_async_copy(x_h.at[st[0]], xr, st[7]).wait()
        if pm == 1:
            pltpu.make_async_copy(pp_h.at[st[0]], pr, st[7]).wait()
            pltpu.make_async_copy(pp_h.at[st[2]], dr, st[7]).wait()

        def edge(i, _):
            w = ewv[pl.ds(i, 16)][0]
            if pm == 1:
                ps = pr[i, pl.ds(0, 16)]
                pd = dr[i, pl.ds(0, 16)]
                wv = jnp.where(ps == pd, w, 0.0)
            else:
                wv = jnp.full((16,), 1.0, F32) * w
            for r in range(8):
                sl = pl.ds(r * 16, 16)
                gbuf[i, sl] = xr[i, sl] * wv
            return 0
        lax.fori_loop(0, K, edge, 0)
        pltpu.sync_copy(gbuf, acc_s.at[db], add=True)

    for t in range(T):
        off = jnp.full((16,), t * N, jnp.int32)
        for pm in range(2):
            def zb(i, _):
                for r in range(8):
                    gbuf[i, pl.ds(r * 16, 16)] = jnp.zeros((16,), F32)
                return 0
            lax.fori_loop(0, K, zb, 0)
            for j in range(rows // K):
                pltpu.sync_copy(gbuf, acc_s.at[pl.ds(r0 + j * K, K)])
            plsc.subcore_barrier()

            loadidx(t, 0, sets[0], off, pm)

            def pair(c2, _):
                c0 = 2 * c2
                loadidx(t, c0 + 1, sets[1], off, pm)
                compute(sets[0], pm)

                @pl.when(c0 + 2 < NCH)
                def _():
                    loadidx(t, c0 + 2, sets[0], off, pm)
                compute(sets[1], pm)
                return 0
            lax.fori_loop(0, NCH // 2, pair, 0)

            plsc.subcore_barrier()
            out_h = aggg_h if pm == 0 else aggm_h
            pltpu.sync_copy(acc_s.at[pl.ds(r0, rows)],
                            out_h.at[pl.ds(cid * TN + t * N + r0, rows)])
            plsc.subcore_barrier()


def _prop_scratch():
    per_set = [
        pltpu.VMEM((K,), jnp.int32),
        pltpu.VMEM((K,), jnp.int32),
        pltpu.VMEM((K,), jnp.int32),
        pltpu.VMEM((K + 16,), F32),
        pltpu.VMEM((K, D), F32),
        pltpu.VMEM((K, D), jnp.int32),
        pltpu.VMEM((K, D), jnp.int32),
    ]
    return ([pltpu.VMEM_SHARED((N, D), F32),
             pltpu.VMEM((K, D), F32)]
            + per_set + per_set
            + [pltpu.SemaphoreType.DMA, pltpu.SemaphoreType.DMA])


_prop = pl.kernel(
    _prop_body,
    out_type=[jax.ShapeDtypeStruct((2 * TN, D), F32),
              jax.ShapeDtypeStruct((2 * TN, D), F32)],
    mesh=_mesh,
    scratch_types=_prop_scratch(),
)


def _proj_body(x_r, w_r, wab_r, hp_r, ab_r, mx_r):
    x = x_r[...]
    hp_r[...] = jnp.dot(x, w_r[...], preferred_element_type=F32)
    ab = jnp.dot(x, wab_r[...], preferred_element_type=F32)
    ab_r[...] = ab
    mx_r[0, 0, :] = jnp.max(ab, axis=0)


def _proj_call(x, w, wab):
    return pl.pallas_call(
        _proj_body,
        grid=(NBT,),
        in_specs=[
            pl.BlockSpec((BR, D), lambda i: (i, 0)),
            pl.BlockSpec((D, D), lambda i: (0, 0)),
            pl.BlockSpec((D, D), lambda i: (0, 0)),
        ],
        out_specs=[
            pl.BlockSpec((BR, D), lambda i: (i, 0)),
            pl.BlockSpec((BR, D), lambda i: (i, 0)),
            pl.BlockSpec((1, 1, D), lambda i: (i, 0, 0)),
        ],
        out_shape=[
            jax.ShapeDtypeStruct((TN, D), F32),
            jax.ShapeDtypeStruct((TN, D), F32),
            jax.ShapeDtypeStruct((NBT, 1, D), F32),
        ],
        interpret=False,
    )(x, w, wab)


def _gpost_body(agg_r, den_r, out_r):
    agg = agg_r[0] + agg_r[1]
    den = den_r[0] + den_r[1] + 1e-16
    d8 = den[:, 0:8]
    dfull = jnp.concatenate([d8] * 16, axis=1)
    q = agg / dfull
    out_r[...] = jnp.where(q > 0, q, jnp.exp(jnp.minimum(q, 0.0)) - 1.0)


def _gpost_call(aggP, denP):
    return pl.pallas_call(
        _gpost_body,
        grid=(NBT,),
        in_specs=[
            pl.BlockSpec((2, BR, D), lambda i: (0, i, 0)),
            pl.BlockSpec((2, BR, 16), lambda i: (0, i, 0)),
        ],
        out_specs=pl.BlockSpec((BR, D), lambda i: (i, 0)),
        out_shape=jax.ShapeDtypeStruct((TN, D), F32),
        interpret=False,
    )(aggP, denP)


def _pools_body(x_r, aggg_r, aggm_r, part_r, wpg_r, wsg_r, wpm_r, wsm_r,
                gstat_r, msum_r, mmax_r, cnt_r):
    i = pl.program_id(1)
    x = x_r[...]
    pgl = x + aggg_r[0, 0] + aggg_r[1, 0]
    hg = jnp.maximum(jnp.dot(pgl, wpg_r[...], preferred_element_type=F32), 0.0)
    scg = jax.nn.sigmoid(jnp.sum(hg * wsg_r[...], axis=1, keepdims=True))
    hgg = hg * scg

    pme = x + aggm_r[0, 0] + aggm_r[1, 0]
    hm = jnp.maximum(jnp.dot(pme, wpm_r[...], preferred_element_type=F32), 0.0)
    scm = jax.nn.sigmoid(jnp.sum(hm * wsm_r[...], axis=1, keepdims=True))
    hgm = hm * scm

    pv = part_r[0, 0, :]
    oh = (pv[:, None] == lax.broadcasted_iota(jnp.int32, (BR, C), 1)).astype(F32)
    gs = jnp.sum(hgg, axis=0)
    gm = jnp.max(hgg, axis=0)
    ms = lax.dot_general(oh, hgm, (((0,), (0,)), ((), ())),
                         preferred_element_type=F32)
    mm = jnp.stack([jnp.where(pv[:, None] == c, hgm, -jnp.inf).max(axis=0)
                    for c in range(C)], axis=0)
    cb = jnp.broadcast_to(jnp.sum(oh, axis=0)[:, None], (C, PH))

    @pl.when(i == 0)
    def _():
        gstat_r[0, 0, :] = gs
        gstat_r[0, 1, :] = gm
        msum_r[0] = ms
        mmax_r[0] = mm
        cnt_r[0] = cb

    @pl.when(i > 0)
    def _():
        gstat_r[0, 0, :] = gstat_r[0, 0, :] + gs
        gstat_r[0, 1, :] = jnp.maximum(gstat_r[0, 1, :], gm)
        msum_r[0] = msum_r[0] + ms
        mmax_r[0] = jnp.maximum(mmax_r[0], mm)
        cnt_r[0] = cnt_r[0] + cb


def _pools_call(x, aggG, aggM, part3, wpg, wsg, wpm, wsm):
    return pl.pallas_call(
        _pools_body,
        grid=(T, NB),
        in_specs=[
            pl.BlockSpec((BR, D), lambda t, i: (t * NB + i, 0)),
            pl.BlockSpec((2, 1, BR, D), lambda t, i: (0, t, i, 0)),
            pl.BlockSpec((2, 1, BR, D), lambda t, i: (0, t, i, 0)),
            pl.BlockSpec((1, 1, BR), lambda t, i: (t * NB + i, 0, 0)),
            pl.BlockSpec((D, PH), lambda t, i: (0, 0)),
            pl.BlockSpec((1, PH), lambda t, i: (0, 0)),
            pl.BlockSpec((D, PH), lambda t, i: (0, 0)),
            pl.BlockSpec((1, PH), lambda t, i: (0, 0)),
        ],
        out_specs=[
            pl.BlockSpec((1, C, PH), lambda t, i: (t, 0, 0)),
            pl.BlockSpec((1, C, PH), lambda t, i: (t, 0, 0)),
            pl.BlockSpec((1, C, PH), lambda t, i: (t, 0, 0)),
            pl.BlockSpec((1, C, PH), lambda t, i: (t, 0, 0)),
        ],
        out_shape=[
            jax.ShapeDtypeStruct((T, C, PH), F32),
            jax.ShapeDtypeStruct((T, C, PH), F32),
            jax.ShapeDtypeStruct((T, C, PH), F32),
            jax.ShapeDtypeStruct((T, C, PH), F32),
        ],
        interpret=False,
    )(x, aggG, aggM, part3, wpg, wsg, wpm, wsm)


def _fuse_body(x_r, part_r, pm_r, mac_r, wmi_r, wme_r, wma_r, qv_r, out_r):
    x = x_r[...]
    h1 = jnp.tanh(jnp.dot(x, wmi_r[...], preferred_element_type=F32))
    pv = part_r[0, 0, :]
    oh = (pv[:, None] == lax.broadcasted_iota(jnp.int32, (BR, C), 1)).astype(F32)
    mfeat = jnp.dot(oh, pm_r[0], preferred_element_type=F32)
    h2 = jnp.tanh(jnp.dot(mfeat, wme_r[...], preferred_element_type=F32))
    h3 = jnp.tanh(jnp.dot(mac_r[0], wma_r[...], preferred_element_type=F32))
    qv = qv_r[...]
    s1 = jnp.sum(h1 * qv, axis=1, keepdims=True)
    s2 = jnp.sum(h2 * qv, axis=1, keepdims=True)
    s3 = jnp.sum(h3 * qv, axis=1, keepdims=True)
    m = jnp.maximum(jnp.maximum(s1, s2), s3)
    e1 = jnp.exp(s1 - m)
    e2 = jnp.exp(s2 - m)
    e3 = jnp.exp(s3 - m)
    den = e1 + e2 + e3
    out_r[...] = (e1 * h1 + e2 * h2 + e3 * h3) / den


def _fuse_call(x, part3, pooledM, macro, wmi, wme, wma, qv):
    return pl.pallas_call(
        _fuse_body,
        grid=(T, NB),
        in_specs=[
            pl.BlockSpec((BR, D), lambda t, i: (t * NB + i, 0)),
            pl.BlockSpec((1, 1, BR), lambda t, i: (t * NB + i, 0, 0)),
            pl.BlockSpec((1, C, D), lambda t, i: (t, 0, 0)),
            pl.BlockSpec((1, 1, D), lambda t, i: (t, 0, 0)),
            pl.BlockSpec((D, D), lambda t, i: (0, 0)),
            pl.BlockSpec((D, D), lambda t, i: (0, 0)),
            pl.BlockSpec((D, D), lambda t, i: (0, 0)),
            pl.BlockSpec((1, D), lambda t, i: (0, 0)),
        ],
        out_specs=pl.BlockSpec((BR, D), lambda t, i: (t * NB + i, 0)),
        out_shape=jax.ShapeDtypeStruct((TN, D), F32),
        interpret=False,
    )(x, part3, pooledM, macro, wmi, wme, wma, qv)


def _lstm_body(a_r, wih_r, whh_r, b_r, ew_r, eb_r, sw_r, sb_r,
               emb_r, scal_r, csq_r):
    h = jnp.zeros((BR, D), F32)
    c = jnp.zeros((BR, D), F32)
    for t in range(T):
        xt = a_r[t]
        z = (jnp.dot(xt, wih_r[...], preferred_element_type=F32)
             + jnp.dot(h, whh_r[...], preferred_element_type=F32) + b_r[...])
        ii = jax.nn.sigmoid(z[:, 0:D])
        ff = jax.nn.sigmoid(z[:, D:2 * D])
        gg = jnp.tanh(z[:, 2 * D:3 * D])
        oo = jax.nn.sigmoid(z[:, 3 * D:4 * D])
        c = ff * c + ii * gg
        h = oo * jnp.tanh(c)
    emb = jnp.tanh(jnp.dot(h, ew_r[...], preferred_element_type=F32) + eb_r[...])
    scal = jax.nn.sigmoid(jnp.dot(h, sw_r[...], preferred_element_type=F32)
                          + sb_r[...])
    emb_r[...] = emb
    scal_r[...] = scal
    csq_r[0, 0, :] = jnp.sum(emb * emb, axis=0)


def _lstm_call(aggs3, wih, whh, b, ew, eb, sw, sb):
    return pl.pallas_call(
        _lstm_body,
        grid=(NB,),
        in_specs=[
            pl.BlockSpec((T, BR, D), lambda i: (0, i, 0)),
            pl.BlockSpec((D, 4 * D), lambda i: (0, 0)),
            pl.BlockSpec((D, 4 * D), lambda i: (0, 0)),
            pl.BlockSpec((1, 4 * D), lambda i: (0, 0)),
            pl.BlockSpec((D, D), lambda i: (0, 0)),
            pl.BlockSpec((1, D), lambda i: (0, 0)),
            pl.BlockSpec((D, D), lambda i: (0, 0)),
            pl.BlockSpec((1, D), lambda i: (0, 0)),
        ],
        out_specs=[
            pl.BlockSpec((BR, D), lambda i: (i, 0)),
            pl.BlockSpec((BR, D), lambda i: (i, 0)),
            pl.BlockSpec((1, 1, D), lambda i: (i, 0, 0)),
        ],
        out_shape=[
            jax.ShapeDtypeStruct((N, D), F32),
            jax.ShapeDtypeStruct((N, D), F32),
            jax.ShapeDtypeStruct((NB, 1, D), F32),
        ],
        interpret=False,
    )(aggs3, wih, whh, b, ew, eb, sw, sb)


def _dec_body(e1_r, e2_r, s1_r, s2_r, inv_r, out_r):
    inv = inv_r[...]
    ei = e1_r[...] * inv
    ej = e2_r[...] * inv
    sqi = jnp.sum(ei * ei, axis=1, keepdims=True)
    sqjr = lax.dot_general(jnp.ones((1, D), F32), ej * ej,
                           (((1,), (1,)), ((), ())), preferred_element_type=F32)
    g = lax.dot_general(ei, ej, (((1,), (1,)), ((), ())),
                        preferred_element_type=F32)
    s = lax.dot_general(s1_r[...], s2_r[...], (((1,), (1,)), ((), ())),
                        preferred_element_type=F32)
    dist2 = sqi + sqjr - 2.0 * g
    out_r[...] = 1.0 + jnp.tanh(-dist2 * s)


def _dec_call(emb, scal, invn):
    nb2 = N // BR2
    return pl.pallas_call(
        _dec_body,
        grid=(nb2, nb2),
        in_specs=[
            pl.BlockSpec((BR2, D), lambda i, j: (i, 0)),
            pl.BlockSpec((BR2, D), lambda i, j: (j, 0)),
            pl.BlockSpec((BR2, D), lambda i, j: (i, 0)),
            pl.BlockSpec((BR2, D), lambda i, j: (j, 0)),
            pl.BlockSpec((1, D), lambda i, j: (0, 0)),
        ],
        out_specs=pl.BlockSpec((BR2, BR2), lambda i, j: (i, j)),
        out_shape=jax.ShapeDtypeStruct((N, N), F32),
        interpret=False,
    )(emb, emb, scal, scal, invn)


def _amat_dup(a):
    blk = (a.T[:, :, None] * jnp.eye(NH, dtype=F32)[None]).reshape(D, NH)
    return jnp.concatenate([blk, blk], axis=1)


def _gat_layer(x, w, asrc, adst, src, dst, eww):
    wab = jnp.concatenate([_amat_dup(asrc), _amat_dup(adst)], axis=1)
    wab = jnp.pad(jnp.dot(w, wab), ((0, 0), (0, D - 32)))
    hp, ab, mx = _proj_call(x, w, wab)
    mxr = jnp.max(mx.reshape(T, NB, D), axis=1)
    msum = mxr[:, 0:16] + mxr[:, 16:32]
    m3 = jnp.where(msum > 0, msum, 0.2 * msum)
    m3pad = jnp.pad(m3, ((0, 8 - T), (0, D - 16)))
    aggP, denP = _gat_edge(hp, ab, m3pad, src, dst, eww)
    denU = denP.reshape(2, T, NP, 8, 16).reshape(2, TN, 16)
    return _gpost_call(aggP.reshape(2, TN, D), denU)


def kernel(feat, edge_weight, params, edge_index, partition):
    p = params
    feat2 = feat.reshape(TN, D)
    src = edge_index[:, 0, :].reshape(TE)
    dst = edge_index[:, 1, :].reshape(TE)
    eww = edge_weight.reshape(TE)
    partp = jnp.broadcast_to(partition.reshape(TN, 1), (TN, D))
    part3 = partition.reshape(T * NB, 1, BR)

    w0 = p['gat0_W'][:, _PERM]
    w1 = p['gat1_W'][_PERM][:, _PERM]
    x1 = _gat_layer(feat2, w0, p['gat0_asrc'], p['gat0_adst'], src, dst, eww)
    x2 = _gat_layer(x1, w1, p['gat1_asrc'], p['gat1_adst'], src, dst, eww)

    aggG, aggM = _prop(x2, partp, src, dst, eww)
    aggG = aggG.reshape(2, T, N, D)
    aggM = aggM.reshape(2, T, N, D)

    gstat, msum, mmax, cnt = _pools_call(
        x2, aggG, aggM, part3,
        p['macro_Wp'][_PERM], p['macro_ws'].reshape(1, PH),
        p['meso_Wp'][_PERM], p['meso_ws'].reshape(1, PH))

    cntv = cnt[:, :, 0]
    mean_c = msum / jnp.maximum(cntv, 1.0)[:, :, None]
    max_c = jnp.where(cntv[:, :, None] > 0, mmax, 0.0)
    pooledM = jnp.concatenate([mean_c, max_c], axis=2)
    macro = jnp.concatenate([gstat[:, 0, :] / N, gstat[:, 1, :]],
                            axis=1).reshape(T, 1, 2 * PH)

    aggs = _fuse_call(x2, part3, pooledM, macro,
                      p['agg_Wmi'][_PERM], p['agg_Wme'], p['agg_Wma'],
                      p['agg_q'].reshape(1, D))

    emb, scal, csq = _lstm_call(
        aggs.reshape(T, N, D), p['lstm_Wih'], p['lstm_Whh'],
        p['lstm_b'].reshape(1, 4 * D), p['emb_W'], p['emb_b'].reshape(1, D),
        p['scal_W'], p['scal_b'].reshape(1, D))

    colnorm = jnp.sqrt(jnp.sum(csq[:, 0, :], axis=0))
    invn = (1.0 / jnp.maximum(colnorm, 1e-12)).reshape(1, D)
    return _dec_call(emb, scal, invn)


# speedup vs baseline: 1.3331x; 1.3331x over previous
import functools

import numpy as np
import jax
import jax.numpy as jnp
from jax import lax
from jax.experimental import pallas as pl
from jax.experimental.pallas import tpu as pltpu
from jax.experimental.pallas import tpu_sc as plsc

N = 4096; E = 65536; T = 3; C = 8; D = 128; NH = 8; DH = 16; PH = 64
TN = T * N
TE = T * E
NC = 2
NS = 16
NW = NC * NS
K = 128
EPW = E // NW
NCH = EPW // K
BR = 256
NB = N // BR
NBT = TN // BR
BR2 = 512
F32 = jnp.float32

_PERM = np.array([(k % NH) * DH + k // NH for k in range(D)], dtype=np.int32)

_mesh = plsc.VectorSubcoreMesh(core_axis_name="c", subcore_axis_name="s",
                               num_cores=NC, num_subcores=NS)


NP = N // 8
KG = 64
NCHG = EPW // KG


def _gat_edge_body(hp_h, abt_h, m3_h, src_h, dst_h, ew_h, aggo_h, deno_h,
                   agg_s, den_s, m3b, abt, exb,
                   sb0, sx0, db0, dx0, dp0, ew0, hr0,
                   sb1, sx1, db1, dx1, dp1, ew1, hr1,
                   sem0, sem1):
    cid = lax.axis_index("c")
    sid = lax.axis_index("s")
    wid = sid * NC + cid
    rows = N // NS
    r0 = sid * rows
    prow = sid * (NP // NS)
    sets = ((sb0, sx0, db0, dx0, dp0, ew0, hr0, sem0),
            (sb1, sx1, db1, dx1, dp1, ew1, hr1, sem1))

    pltpu.sync_copy(m3_h, m3b)

    def loadidx(t, ch, st, off):
        sb, sx, db, dx, dp, ewv = st[0], st[1], st[2], st[3], st[4], st[5]
        base = t * E + wid * EPW + ch * KG
        pltpu.sync_copy(src_h.at[pl.ds(base, KG)], sb)
        pltpu.sync_copy(src_h.at[pl.ds(base, KG)], sx.at[pl.ds(0, KG)])
        pltpu.sync_copy(dst_h.at[pl.ds(base, KG)], db)
        pltpu.sync_copy(dst_h.at[pl.ds(base, KG)], dx.at[pl.ds(0, KG)])
        pltpu.sync_copy(ew_h.at[pl.ds(base, KG)], ewv.at[pl.ds(0, KG)])
        for j in range(KG // 16):
            sl = pl.ds(j * 16, 16)
            sb[sl] = sb[sl] + off
            dp[sl] = lax.shift_right_logical(db[sl], 3)
        pltpu.async_copy(hp_h.at[sb], st[6], st[7])

    def compute(st, mv):
        sx, db, dx, dp, ewv, hr = st[1], st[2], st[3], st[4], st[5], st[6]
        pltpu.make_async_copy(hp_h.at[st[0]], hr, st[7]).wait()

        @functools.partial(plsc.parallel_loop, 0, KG, unroll=1)
        def edge(i):
            cola = lax.iota(jnp.int32, 16) & 7
            s = sx[pl.ds(i, 16)][0]
            dl = dx[pl.ds(i, 16)][0]
            va = plsc.load_gather(abt, [s * 16 + cola])
            vb = plsc.load_gather(abt, [dl * 16 + cola + 8])
            e = va + vb
            e = jnp.where(e > 0, e, 0.2 * e)
            ex = jnp.exp(e - mv) * ewv[pl.ds(i, 16)][0]
            loff = (dl & 7) * 16
            z = jnp.zeros((16,), F32)
            for g in range(8):
                exb[i, pl.ds(g * 16, 16)] = jnp.where(loff == g * 16, ex, z)
            for r in range(8):
                sl = pl.ds(r * 16, 16)
                hr[i, sl] = hr[i, sl] * ex

        pltpu.sync_copy(hr, agg_s.at[db], add=True)
        pltpu.sync_copy(exb, den_s.at[dp], add=True)

    for t in range(T):
        pltpu.sync_copy(abt_h.at[pl.ds(t * N * 16, N * 16)], abt)

        def zb(i, _):
            for r in range(8):
                hr0[i, pl.ds(r * 16, 16)] = jnp.zeros((16,), F32)
            return 0
        lax.fori_loop(0, KG, zb, 0)
        for j in range(rows // KG):
            pltpu.sync_copy(hr0, agg_s.at[pl.ds(r0 + j * KG, KG)])
        pltpu.sync_copy(hr0.at[pl.ds(0, NP // NS)],
                        den_s.at[pl.ds(prow, NP // NS)])
        plsc.subcore_barrier()

        mv = m3b[t, pl.ds(0, 16)]
        off = jnp.full((16,), t * N, jnp.int32)
        loadidx(t, 0, sets[0], off)

        def pair(c2, _):
            c0 = 2 * c2
            loadidx(t, c0 + 1, sets[1], off)
            compute(sets[0], mv)

            @pl.when(c0 + 2 < NCHG)
            def _():
                loadidx(t, c0 + 2, sets[0], off)
            compute(sets[1], mv)
            return 0
        lax.fori_loop(0, NCHG // 2, pair, 0)

        plsc.subcore_barrier()
        pltpu.sync_copy(agg_s.at[pl.ds(r0, rows)],
                        aggo_h.at[pl.ds(cid * TN + t * N + r0, rows)])
        pltpu.sync_copy(den_s.at[pl.ds(prow, NP // NS)],
                        deno_h.at[pl.ds(cid * T * NP + t * NP + prow,
                                        NP // NS)])
        plsc.subcore_barrier()


def _gat_scratch():
    per_set = [
        pltpu.VMEM((KG,), jnp.int32),
        pltpu.VMEM((KG + 16,), jnp.int32),
        pltpu.VMEM((KG,), jnp.int32),
        pltpu.VMEM((KG + 16,), jnp.int32),
        pltpu.VMEM((KG,), jnp.int32),
        pltpu.VMEM((KG + 16,), F32),
        pltpu.VMEM((KG, D), F32),
    ]
    return ([pltpu.VMEM_SHARED((N, D), F32),
             pltpu.VMEM_SHARED((NP, D), F32),
             pltpu.VMEM((8, D), F32),
             pltpu.VMEM((N * 16,), F32),
             pltpu.VMEM((KG, D), F32)]
            + per_set + per_set
            + [pltpu.SemaphoreType.DMA, pltpu.SemaphoreType.DMA])


_gat_edge = pl.kernel(
    _gat_edge_body,
    out_type=[jax.ShapeDtypeStruct((2 * TN, D), F32),
              jax.ShapeDtypeStruct((2 * T * NP, D), F32)],
    mesh=_mesh,
    scratch_types=_gat_scratch(),
)


def _prop_body(x_h, pp_h, src_h, dst_h, ew_h, aggg_h, aggm_h,
               acc_s, gbuf, ptb,
               sb0, sx0, db0, dx0, ew0, xr0,
               sb1, sx1, db1, dx1, ew1, xr1,
               sem0, sem1):
    cid = lax.axis_index("c")
    sid = lax.axis_index("s")
    wid = sid * NC + cid
    rows = N // NS
    r0 = sid * rows
    sets = ((sb0, sx0, db0, dx0, ew0, xr0, sem0),
            (sb1, sx1, db1, dx1, ew1, xr1, sem1))

    def loadidx(t, ch, st, off):
        sb, sx, db, dx, ewv = st[0], st[1], st[2], st[3], st[4]
        base = t * E + wid * EPW + ch * K
        pltpu.sync_copy(src_h.at[pl.ds(base, K)], sb)
        pltpu.sync_copy(src_h.at[pl.ds(base, K)], sx.at[pl.ds(0, K)])
        pltpu.sync_copy(dst_h.at[pl.ds(base, K)], db)
        pltpu.sync_copy(dst_h.at[pl.ds(base, K)], dx.at[pl.ds(0, K)])
        pltpu.sync_copy(ew_h.at[pl.ds(base, K)], ewv.at[pl.ds(0, K)])
        for j in range(K // 16):
            sl = pl.ds(j * 16, 16)
            sb[sl] = sb[sl] + off
        pltpu.async_copy(x_h.at[sb], st[5], st[6])

    def compute(st, pm):
        sx, db, dx, ewv, xr = st[1], st[2], st[3], st[4], st[5]
        pltpu.make_async_copy(x_h.at[st[0]], xr, st[6]).wait()

        @functools.partial(plsc.parallel_loop, 0, K, unroll=1)
        def edge(i):
            w = ewv[pl.ds(i, 16)][0]
            if pm == 1:
                s = sx[pl.ds(i, 16)][0]
                dl = dx[pl.ds(i, 16)][0]
                ps = plsc.load_gather(ptb, [jnp.full((16,), s, jnp.int32)])
                pd = plsc.load_gather(ptb, [jnp.full((16,), dl, jnp.int32)])
                wv = jnp.where(ps == pd, w, 0.0)
            else:
                wv = jnp.full((16,), 1.0, F32) * w
            for r in range(8):
                sl = pl.ds(r * 16, 16)
                gbuf[i, sl] = xr[i, sl] * wv

        pltpu.sync_copy(gbuf, acc_s.at[db], add=True)

    for t in range(T):
        off = jnp.full((16,), t * N, jnp.int32)
        pltpu.sync_copy(pp_h.at[pl.ds(t * N, N)], ptb)
        for pm in range(2):
            def zb(i, _):
                for r in range(8):
                    gbuf[i, pl.ds(r * 16, 16)] = jnp.zeros((16,), F32)
                return 0
            lax.fori_loop(0, K, zb, 0)
            for j in range(rows // K):
                pltpu.sync_copy(gbuf, acc_s.at[pl.ds(r0 + j * K, K)])
            plsc.subcore_barrier()

            loadidx(t, 0, sets[0], off)

            def pair(c2, _):
                c0 = 2 * c2
                loadidx(t, c0 + 1, sets[1], off)
                compute(sets[0], pm)

                @pl.when(c0 + 2 < NCH)
                def _():
                    loadidx(t, c0 + 2, sets[0], off)
                compute(sets[1], pm)
                return 0
            lax.fori_loop(0, NCH // 2, pair, 0)

            plsc.subcore_barrier()
            out_h = aggg_h if pm == 0 else aggm_h
            pltpu.sync_copy(acc_s.at[pl.ds(r0, rows)],
                            out_h.at[pl.ds(cid * TN + t * N + r0, rows)])
            plsc.subcore_barrier()


def _prop_scratch():
    per_set = [
        pltpu.VMEM((K,), jnp.int32),
        pltpu.VMEM((K + 16,), jnp.int32),
        pltpu.VMEM((K,), jnp.int32),
        pltpu.VMEM((K + 16,), jnp.int32),
        pltpu.VMEM((K + 16,), F32),
        pltpu.VMEM((K, D), F32),
    ]
    return ([pltpu.VMEM_SHARED((N, D), F32),
             pltpu.VMEM((K, D), F32),
             pltpu.VMEM((N,), jnp.int32)]
            + per_set + per_set
            + [pltpu.SemaphoreType.DMA, pltpu.SemaphoreType.DMA])


_prop = pl.kernel(
    _prop_body,
    out_type=[jax.ShapeDtypeStruct((2 * TN, D), F32),
              jax.ShapeDtypeStruct((2 * TN, D), F32)],
    mesh=_mesh,
    scratch_types=_prop_scratch(),
)


def _proj_body(x_r, w_r, wab_r, hp_r, ab_r, mx_r):
    x = x_r[...]
    hp_r[...] = jnp.dot(x, w_r[...], preferred_element_type=F32)
    ab = jnp.dot(x, wab_r[...], preferred_element_type=F32)
    ab_r[...] = ab
    mx_r[0, 0, :] = jnp.max(ab, axis=0)


def _proj_call(x, w, wab):
    return pl.pallas_call(
        _proj_body,
        grid=(NBT,),
        in_specs=[
            pl.BlockSpec((BR, D), lambda i: (i, 0)),
            pl.BlockSpec((D, D), lambda i: (0, 0)),
            pl.BlockSpec((D, D), lambda i: (0, 0)),
        ],
        out_specs=[
            pl.BlockSpec((BR, D), lambda i: (i, 0)),
            pl.BlockSpec((BR, D), lambda i: (i, 0)),
            pl.BlockSpec((1, 1, D), lambda i: (i, 0, 0)),
        ],
        out_shape=[
            jax.ShapeDtypeStruct((TN, D), F32),
            jax.ShapeDtypeStruct((TN, D), F32),
            jax.ShapeDtypeStruct((NBT, 1, D), F32),
        ],
        interpret=False,
    )(x, w, wab)


def _gpost_body(agg_r, den_r, out_r):
    agg = agg_r[0] + agg_r[1]
    den = den_r[0] + den_r[1] + 1e-16
    d8 = den[:, 0:8]
    dfull = jnp.concatenate([d8] * 16, axis=1)
    q = agg / dfull
    out_r[...] = jnp.where(q > 0, q, jnp.exp(jnp.minimum(q, 0.0)) - 1.0)


def _gpost_call(aggP, denP):
    return pl.pallas_call(
        _gpost_body,
        grid=(NBT,),
        in_specs=[
            pl.BlockSpec((2, BR, D), lambda i: (0, i, 0)),
            pl.BlockSpec((2, BR, 16), lambda i: (0, i, 0)),
        ],
        out_specs=pl.BlockSpec((BR, D), lambda i: (i, 0)),
        out_shape=jax.ShapeDtypeStruct((TN, D), F32),
        interpret=False,
    )(aggP, denP)


def _pools_body(x_r, aggg_r, aggm_r, part_r, wpg_r, wsg_r, wpm_r, wsm_r,
                gstat_r, msum_r, mmax_r, cnt_r):
    i = pl.program_id(1)
    x = x_r[...]
    pgl = x + aggg_r[0, 0] + aggg_r[1, 0]
    hg = jnp.maximum(jnp.dot(pgl, wpg_r[...], preferred_element_type=F32), 0.0)
    scg = jax.nn.sigmoid(jnp.sum(hg * wsg_r[...], axis=1, keepdims=True))
    hgg = hg * scg

    pme = x + aggm_r[0, 0] + aggm_r[1, 0]
    hm = jnp.maximum(jnp.dot(pme, wpm_r[...], preferred_element_type=F32), 0.0)
    scm = jax.nn.sigmoid(jnp.sum(hm * wsm_r[...], axis=1, keepdims=True))
    hgm = hm * scm

    pv = part_r[0, 0, :]
    oh = (pv[:, None] == lax.broadcasted_iota(jnp.int32, (BR, C), 1)).astype(F32)
    gs = jnp.sum(hgg, axis=0)
    gm = jnp.max(hgg, axis=0)
    ms = lax.dot_general(oh, hgm, (((0,), (0,)), ((), ())),
                         preferred_element_type=F32)
    mm = jnp.stack([jnp.where(pv[:, None] == c, hgm, -jnp.inf).max(axis=0)
                    for c in range(C)], axis=0)
    cb = jnp.broadcast_to(jnp.sum(oh, axis=0)[:, None], (C, PH))

    @pl.when(i == 0)
    def _():
        gstat_r[0, 0, :] = gs
        gstat_r[0, 1, :] = gm
        msum_r[0] = ms
        mmax_r[0] = mm
        cnt_r[0] = cb

    @pl.when(i > 0)
    def _():
        gstat_r[0, 0, :] = gstat_r[0, 0, :] + gs
        gstat_r[0, 1, :] = jnp.maximum(gstat_r[0, 1, :], gm)
        msum_r[0] = msum_r[0] + ms
        mmax_r[0] = jnp.maximum(mmax_r[0], mm)
        cnt_r[0] = cnt_r[0] + cb


def _pools_call(x, aggG, aggM, part3, wpg, wsg, wpm, wsm):
    return pl.pallas_call(
        _pools_body,
        grid=(T, NB),
        in_specs=[
            pl.BlockSpec((BR, D), lambda t, i: (t * NB + i, 0)),
            pl.BlockSpec((2, 1, BR, D), lambda t, i: (0, t, i, 0)),
            pl.BlockSpec((2, 1, BR, D), lambda t, i: (0, t, i, 0)),
            pl.BlockSpec((1, 1, BR), lambda t, i: (t * NB + i, 0, 0)),
            pl.BlockSpec((D, PH), lambda t, i: (0, 0)),
            pl.BlockSpec((1, PH), lambda t, i: (0, 0)),
            pl.BlockSpec((D, PH), lambda t, i: (0, 0)),
            pl.BlockSpec((1, PH), lambda t, i: (0, 0)),
        ],
        out_specs=[
            pl.BlockSpec((1, C, PH), lambda t, i: (t, 0, 0)),
            pl.BlockSpec((1, C, PH), lambda t, i: (t, 0, 0)),
            pl.BlockSpec((1, C, PH), lambda t, i: (t, 0, 0)),
            pl.BlockSpec((1, C, PH), lambda t, i: (t, 0, 0)),
        ],
        out_shape=[
            jax.ShapeDtypeStruct((T, C, PH), F32),
            jax.ShapeDtypeStruct((T, C, PH), F32),
            jax.ShapeDtypeStruct((T, C, PH), F32),
            jax.ShapeDtypeStruct((T, C, PH), F32),
        ],
        interpret=False,
    )(x, aggG, aggM, part3, wpg, wsg, wpm, wsm)


def _fuse_body(x_r, part_r, pm_r, mac_r, wmi_r, wme_r, wma_r, qv_r, out_r):
    x = x_r[...]
    h1 = jnp.tanh(jnp.dot(x, wmi_r[...], preferred_element_type=F32))
    pv = part_r[0, 0, :]
    oh = (pv[:, None] == lax.broadcasted_iota(jnp.int32, (BR, C), 1)).astype(F32)
    mfeat = jnp.dot(oh, pm_r[0], preferred_element_type=F32)
    h2 = jnp.tanh(jnp.dot(mfeat, wme_r[...], preferred_element_type=F32))
    h3 = jnp.tanh(jnp.dot(mac_r[0], wma_r[...], preferred_element_type=F32))
    qv = qv_r[...]
    s1 = jnp.sum(h1 * qv, axis=1, keepdims=True)
    s2 = jnp.sum(h2 * qv, axis=1, keepdims=True)
    s3 = jnp.sum(h3 * qv, axis=1, keepdims=True)
    m = jnp.maximum(jnp.maximum(s1, s2), s3)
    e1 = jnp.exp(s1 - m)
    e2 = jnp.exp(s2 - m)
    e3 = jnp.exp(s3 - m)
    den = e1 + e2 + e3
    out_r[...] = (e1 * h1 + e2 * h2 + e3 * h3) / den


def _fuse_call(x, part3, pooledM, macro, wmi, wme, wma, qv):
    return pl.pallas_call(
        _fuse_body,
        grid=(T, NB),
        in_specs=[
            pl.BlockSpec((BR, D), lambda t, i: (t * NB + i, 0)),
            pl.BlockSpec((1, 1, BR), lambda t, i: (t * NB + i, 0, 0)),
            pl.BlockSpec((1, C, D), lambda t, i: (t, 0, 0)),
            pl.BlockSpec((1, 1, D), lambda t, i: (t, 0, 0)),
            pl.BlockSpec((D, D), lambda t, i: (0, 0)),
            pl.BlockSpec((D, D), lambda t, i: (0, 0)),
            pl.BlockSpec((D, D), lambda t, i: (0, 0)),
            pl.BlockSpec((1, D), lambda t, i: (0, 0)),
        ],
        out_specs=pl.BlockSpec((BR, D), lambda t, i: (t * NB + i, 0)),
        out_shape=jax.ShapeDtypeStruct((TN, D), F32),
        interpret=False,
    )(x, part3, pooledM, macro, wmi, wme, wma, qv)


def _lstm_body(a_r, wih_r, whh_r, b_r, ew_r, eb_r, sw_r, sb_r,
               emb_r, scal_r, csq_r):
    h = jnp.zeros((BR, D), F32)
    c = jnp.zeros((BR, D), F32)
    for t in range(T):
        xt = a_r[t]
        z = (jnp.dot(xt, wih_r[...], preferred_element_type=F32)
             + jnp.dot(h, whh_r[...], preferred_element_type=F32) + b_r[...])
        ii = jax.nn.sigmoid(z[:, 0:D])
        ff = jax.nn.sigmoid(z[:, D:2 * D])
        gg = jnp.tanh(z[:, 2 * D:3 * D])
        oo = jax.nn.sigmoid(z[:, 3 * D:4 * D])
        c = ff * c + ii * gg
        h = oo * jnp.tanh(c)
    emb = jnp.tanh(jnp.dot(h, ew_r[...], preferred_element_type=F32) + eb_r[...])
    scal = jax.nn.sigmoid(jnp.dot(h, sw_r[...], preferred_element_type=F32)
                          + sb_r[...])
    emb_r[...] = emb
    scal_r[...] = scal
    csq_r[0, 0, :] = jnp.sum(emb * emb, axis=0)


def _lstm_call(aggs3, wih, whh, b, ew, eb, sw, sb):
    return pl.pallas_call(
        _lstm_body,
        grid=(NB,),
        in_specs=[
            pl.BlockSpec((T, BR, D), lambda i: (0, i, 0)),
            pl.BlockSpec((D, 4 * D), lambda i: (0, 0)),
            pl.BlockSpec((D, 4 * D), lambda i: (0, 0)),
            pl.BlockSpec((1, 4 * D), lambda i: (0, 0)),
            pl.BlockSpec((D, D), lambda i: (0, 0)),
            pl.BlockSpec((1, D), lambda i: (0, 0)),
            pl.BlockSpec((D, D), lambda i: (0, 0)),
            pl.BlockSpec((1, D), lambda i: (0, 0)),
        ],
        out_specs=[
            pl.BlockSpec((BR, D), lambda i: (i, 0)),
            pl.BlockSpec((BR, D), lambda i: (i, 0)),
            pl.BlockSpec((1, 1, D), lambda i: (i, 0, 0)),
        ],
        out_shape=[
            jax.ShapeDtypeStruct((N, D), F32),
            jax.ShapeDtypeStruct((N, D), F32),
            jax.ShapeDtypeStruct((NB, 1, D), F32),
        ],
        interpret=False,
    )(aggs3, wih, whh, b, ew, eb, sw, sb)


def _dec_body(e1_r, e2_r, s1_r, s2_r, inv_r, out_r):
    inv = inv_r[...]
    ei = e1_r[...] * inv
    ej = e2_r[...] * inv
    sqi = jnp.sum(ei * ei, axis=1, keepdims=True)
    sqjr = lax.dot_general(jnp.ones((1, D), F32), ej * ej,
                           (((1,), (1,)), ((), ())), preferred_element_type=F32)
    g = lax.dot_general(ei, ej, (((1,), (1,)), ((), ())),
                        preferred_element_type=F32)
    s = lax.dot_general(s1_r[...], s2_r[...], (((1,), (1,)), ((), ())),
                        preferred_element_type=F32)
    dist2 = sqi + sqjr - 2.0 * g
    out_r[...] = 1.0 + jnp.tanh(-dist2 * s)


def _dec_call(emb, scal, invn):
    nb2 = N // BR2
    return pl.pallas_call(
        _dec_body,
        grid=(nb2, nb2),
        in_specs=[
            pl.BlockSpec((BR2, D), lambda i, j: (i, 0)),
            pl.BlockSpec((BR2, D), lambda i, j: (j, 0)),
            pl.BlockSpec((BR2, D), lambda i, j: (i, 0)),
            pl.BlockSpec((BR2, D), lambda i, j: (j, 0)),
            pl.BlockSpec((1, D), lambda i, j: (0, 0)),
        ],
        out_specs=pl.BlockSpec((BR2, BR2), lambda i, j: (i, j)),
        out_shape=jax.ShapeDtypeStruct((N, N), F32),
        interpret=False,
    )(emb, emb, scal, scal, invn)


def _amat(a):
    return (a.T[:, :, None] * jnp.eye(NH, dtype=F32)[None]).reshape(D, NH)


def _gat_layer(x, w, asrc, adst, src, dst, eww):
    wab = jnp.dot(w, jnp.concatenate([_amat(asrc), _amat(adst)], axis=1))
    wab = jnp.pad(wab, ((0, 0), (0, D - 16)))
    hp, ab, mx = _proj_call(x, w, wab)
    mxr = jnp.max(mx.reshape(T, NB, D), axis=1)
    msum = mxr[:, 0:8] + mxr[:, 8:16]
    m3 = jnp.where(msum > 0, msum, 0.2 * msum)
    m3pad = jnp.pad(jnp.tile(m3, (1, 2)), ((0, 8 - T), (0, D - 16)))
    abt = ab[:, 0:16].reshape(TN * 16)
    aggP, denP = _gat_edge(hp, abt, m3pad, src, dst, eww)
    denU = denP.reshape(2, T, NP, 8, 16).reshape(2, TN, 16)
    return _gpost_call(aggP.reshape(2, TN, D), denU)


def kernel(feat, edge_weight, params, edge_index, partition):
    p = params
    feat2 = feat.reshape(TN, D)
    src = edge_index[:, 0, :].reshape(TE)
    dst = edge_index[:, 1, :].reshape(TE)
    eww = edge_weight.reshape(TE)
    part1 = partition.reshape(TN)
    part3 = partition.reshape(T * NB, 1, BR)

    w0 = p['gat0_W'][:, _PERM]
    w1 = p['gat1_W'][_PERM][:, _PERM]
    x1 = _gat_layer(feat2, w0, p['gat0_asrc'], p['gat0_adst'], src, dst, eww)
    x2 = _gat_layer(x1, w1, p['gat1_asrc'], p['gat1_adst'], src, dst, eww)

    aggG, aggM = _prop(x2, part1, src, dst, eww)
    aggG = aggG.reshape(2, T, N, D)
    aggM = aggM.reshape(2, T, N, D)

    gstat, msum, mmax, cnt = _pools_call(
        x2, aggG, aggM, part3,
        p['macro_Wp'][_PERM], p['macro_ws'].reshape(1, PH),
        p['meso_Wp'][_PERM], p['meso_ws'].reshape(1, PH))

    cntv = cnt[:, :, 0]
    mean_c = msum / jnp.maximum(cntv, 1.0)[:, :, None]
    max_c = jnp.where(cntv[:, :, None] > 0, mmax, 0.0)
    pooledM = jnp.concatenate([mean_c, max_c], axis=2)
    macro = jnp.concatenate([gstat[:, 0, :] / N, gstat[:, 1, :]],
                            axis=1).reshape(T, 1, 2 * PH)

    aggs = _fuse_call(x2, part3, pooledM, macro,
                      p['agg_Wmi'][_PERM], p['agg_Wme'], p['agg_Wma'],
                      p['agg_q'].reshape(1, D))

    emb, scal, csq = _lstm_call(
        aggs.reshape(T, N, D), p['lstm_Wih'], p['lstm_Whh'],
        p['lstm_b'].reshape(1, 4 * D), p['emb_W'], p['emb_b'].reshape(1, D),
        p['scal_W'], p['scal_b'].reshape(1, D))

    colnorm = jnp.sqrt(jnp.sum(csq[:, 0, :], axis=0))
    invn = (1.0 / jnp.maximum(colnorm, 1e-12)).reshape(1, D)
    return _dec_call(emb, scal, invn)
